# baseline (speedup 1.0000x reference)
import jax
import jax.numpy as jnp
from jax import lax
from jax.experimental import pallas as pl
from jax.experimental.pallas import tpu as pltpu

K_WAY = 5
N_SHOT = 5
NSUP = K_WAY * N_SHOT
D = 4096
NQ = 16384
N_AUG = 70
TOPK = 50
NS = K_WAY + N_AUG

QT = 1024
NT1 = NQ // QT
TAIL = 32

_NOISE = jax.random.normal(jax.random.key(42), (K_WAY, N_AUG, D),
                           dtype=jnp.float32)


def _logsumexp_rows(z):
    m = jnp.max(z, axis=1, keepdims=True)
    return m + jnp.log(jnp.sum(jnp.exp(z - m), axis=1, keepdims=True))


def _accumulate(q, tgt, proto, pn, valid, stats_scr, cnt_scr, loss_ref):
    rows = q.shape[0]
    qsq = q * q
    qn = jnp.sum(qsq, axis=1, keepdims=True)
    qp = lax.dot_general(q, proto, (((1,), (1,)), ((), ())),
                         preferred_element_type=jnp.float32)
    d2 = qn + pn[None, :] - 2.0 * qp
    d_e = jnp.sqrt(jnp.clip(d2, 1e-12))

    z = -d_e
    zmax = jnp.max(z, axis=1, keepdims=True)
    iota_k = lax.broadcasted_iota(jnp.int32, (rows, K_WAY), 1)
    cand = jnp.where(z >= zmax, iota_k, K_WAY)
    pred = jnp.min(cand, axis=1, keepdims=True)
    onehot = jnp.where((iota_k == pred) & valid, 1.0, 0.0)

    cnt_scr[...] += jnp.sum(onehot, axis=0, keepdims=True)
    oh_bf = onehot.astype(jnp.bfloat16)
    stats_scr[0] += lax.dot_general(oh_bf, q.astype(jnp.bfloat16),
                                    (((0,), (0,)), ((), ())),
                                    preferred_element_type=jnp.float32)
    stats_scr[1] += lax.dot_general(oh_bf, qsq.astype(jnp.bfloat16),
                                    (((0,), (0,)), ((), ())),
                                    preferred_element_type=jnp.float32)

    a = z - _logsumexp_rows(z)
    b = a - _logsumexp_rows(a)
    tgt_f = tgt.astype(jnp.float32)
    loss_ref[...] += -jnp.sum(tgt_f * b, keepdims=True).reshape(1, 1)


def _refresh_protos(sup_ref, stats_scr, cnt_scr, noise_ref, pnew_scr):
    sup = sup_ref[:NSUP, :].reshape(K_WAY, N_SHOT, D)
    proto = jnp.mean(sup, axis=1)
    cnt = cnt_scr[...].reshape(K_WAY, 1) + float(N_SHOT)
    s_sum = jnp.sum(sup, axis=1) + stats_scr[0]
    s_sq = jnp.sum(sup * sup, axis=1) + stats_scr[1]
    mean_c = s_sum / cnt
    var_c = (s_sq - cnt * mean_c * mean_c) / (cnt - 1.0)
    std_c = jnp.sqrt(jnp.clip(var_c, 1e-12))

    noise = noise_ref[...]
    delta = mean_c - proto
    u = delta * std_c
    v = std_c * std_c
    dn2 = jnp.sum(delta * delta, axis=1, keepdims=True)
    bdims = (((2,), (1,)), ((0,), (0,)))
    cross = lax.dot_general(noise, u, bdims,
                            preferred_element_type=jnp.float32)
    quad = lax.dot_general(noise * noise, v, bdims,
                           preferred_element_type=jnp.float32)
    d2_smp = dn2 + 2.0 * cross + quad

    dsup = sup - proto[:, None, :]
    d2_sup = jnp.sum(dsup * dsup, axis=2)
    d = jnp.sqrt(jnp.clip(jnp.concatenate([d2_sup, d2_smp], axis=1),
                          1e-12))

    di = d[:, :, None]
    dj = d[:, None, :]
    ii = lax.broadcasted_iota(jnp.int32, (K_WAY, NS, NS), 1)
    jj = lax.broadcasted_iota(jnp.int32, (K_WAY, NS, NS), 2)
    cmp = (dj < di) | ((dj == di) & (jj < ii))
    rank = jnp.sum(cmp.astype(jnp.float32), axis=2)
    sel = (rank < float(TOPK)).astype(jnp.float32)
    sel_sup = sel[:, :N_SHOT]
    sel_smp = sel[:, N_SHOT:]
    nsel = jnp.sum(sel_smp, axis=1, keepdims=True)

    sup_part = lax.dot_general(sel_sup, sup, (((1,), (1,)), ((0,), (0,))),
                               preferred_element_type=jnp.float32)
    noise_part = lax.dot_general(sel_smp, noise, (((1,), (1,)), ((0,), (0,))),
                                 preferred_element_type=jnp.float32)
    pnew = sup_part + nsel * mean_c + std_c * noise_part
    pnew_scr[:K_WAY] = pnew * (1.0 / TOPK)


def _softmax_rows(q, pnew, pn):
    qn = jnp.sum(q * q, axis=1, keepdims=True)
    qp = lax.dot_general(q, pnew, (((1,), (1,)), ((), ())),
                         preferred_element_type=jnp.float32)
    d2 = qn + pn[None, :] - 2.0 * qp
    z = -jnp.sqrt(jnp.clip(d2, 1e-12))
    m = jnp.max(z, axis=1, keepdims=True)
    e = jnp.exp(z - m)
    return e / jnp.sum(e, axis=1, keepdims=True)


def _body(q_ref, tailf_ref, tgt_ref, tgtt_ref, noise_ref,
          loss_ref, y_ref, yt_ref,
          proto_scr, stats_scr, cnt_scr, pnew_scr):
    i = pl.program_id(0)

    @pl.when(i == 0)
    def _init():
        stats_scr[...] = jnp.zeros_like(stats_scr)
        cnt_scr[...] = jnp.zeros_like(cnt_scr)
        loss_ref[...] = jnp.zeros_like(loss_ref)
        sup = q_ref[:NSUP, :].reshape(K_WAY, N_SHOT, D)
        proto_scr[:K_WAY] = jnp.mean(sup, axis=1)

    @pl.when(i < NT1)
    def _phase1():
        proto = proto_scr[:K_WAY]
        pn = jnp.sum(proto * proto, axis=1)
        q = q_ref[...]
        r0 = lax.broadcasted_iota(jnp.int32, (QT, 1), 0) + i * QT
        valid = r0 >= NSUP
        _accumulate(q, tgt_ref[...], proto, pn, valid,
                    stats_scr, cnt_scr, loss_ref)

        @pl.when(i == NT1 - 1)
        def _tail():
            rloc = lax.broadcasted_iota(jnp.int32, (TAIL, 1), 0)
            tvalid = rloc < NSUP
            qt = jnp.where(tvalid, tailf_ref[...], 0.0)
            _accumulate(qt, tgtt_ref[...], proto, pn, tvalid,
                        stats_scr, cnt_scr, loss_ref)

    @pl.when(i == NT1)
    def _mid():
        _refresh_protos(q_ref, stats_scr, cnt_scr, noise_ref, pnew_scr)

    @pl.when(i >= NT1)
    def _phase2():
        pnew = pnew_scr[:K_WAY]
        pn = jnp.sum(pnew * pnew, axis=1)
        y_ref[...] = _softmax_rows(q_ref[...], pnew, pn)

        @pl.when(i == 2 * NT1 - 1)
        def _tail2():
            yt_ref[...] = _softmax_rows(tailf_ref[...], pnew, pn)


@jax.jit
def kernel(feat, label):
    tgt_oh = jax.nn.one_hot(label[1], K_WAY, dtype=jnp.int8)
    tgt_pad = jnp.pad(tgt_oh, ((NSUP, TAIL - NSUP), (0, 0)))

    loss_sum, y_main, y_tail = pl.pallas_call(
        _body,
        grid=(2 * NT1,),
        in_specs=[
            pl.BlockSpec((QT, D),
                         lambda i: (jnp.where(i < NT1, i, i - NT1), 0)),
            pl.BlockSpec((TAIL, D), lambda i: (NQ // TAIL, 0)),
            pl.BlockSpec((QT, K_WAY), lambda i: (jnp.minimum(i, NT1 - 1), 0)),
            pl.BlockSpec((TAIL, K_WAY), lambda i: (NQ // TAIL, 0)),
            pl.BlockSpec((K_WAY, N_AUG, D), lambda i: (0, 0, 0)),
        ],
        out_specs=[
            pl.BlockSpec((1, 1), lambda i: (0, 0)),
            pl.BlockSpec((QT, K_WAY),
                         lambda i: (jnp.where(i < NT1, 0, i - NT1), 0)),
            pl.BlockSpec((TAIL, K_WAY), lambda i: (0, 0)),
        ],
        out_shape=[
            jax.ShapeDtypeStruct((1, 1), jnp.float32),
            jax.ShapeDtypeStruct((NQ, K_WAY), jnp.float32),
            jax.ShapeDtypeStruct((TAIL, K_WAY), jnp.float32),
        ],
        scratch_shapes=[
            pltpu.VMEM((8, D), jnp.float32),
            pltpu.VMEM((2, K_WAY, D), jnp.float32),
            pltpu.VMEM((1, K_WAY), jnp.float32),
            pltpu.VMEM((8, D), jnp.float32),
        ],
    )(feat, feat, tgt_pad, tgt_pad, _NOISE)

    y_pred = jnp.concatenate(
        [lax.slice(y_main, (NSUP, 0), (NQ, K_WAY)),
         lax.slice(y_tail, (0, 0), (NSUP, K_WAY))], axis=0)
    loss = loss_sum[0, 0] / NQ
    return (y_pred, loss)


# speedup vs baseline: 1.2654x; 1.2654x over previous
import jax
import jax.numpy as jnp
from jax import lax
from jax.experimental import pallas as pl
from jax.experimental.pallas import tpu as pltpu

K_WAY = 5
N_SHOT = 5
NSUP = K_WAY * N_SHOT
D = 4096
NQ = 16384
N_AUG = 70
TOPK = 50
NS = K_WAY + N_AUG

QT = 1024
NT1 = NQ // QT
TAIL = 32

_NOISE = jax.random.normal(jax.random.key(42), (K_WAY, N_AUG, D),
                           dtype=jnp.float32)


def _logsumexp_rows(z):
    m = jnp.max(z, axis=1, keepdims=True)
    return m + jnp.log(jnp.sum(jnp.exp(z - m), axis=1, keepdims=True))


def _accumulate(q, tgt, proto, pn, valid, stats_scr, cnt_scr, loss_ref):
    rows = q.shape[0]
    qsq = q * q
    qn = jnp.sum(qsq, axis=1, keepdims=True)
    qp = lax.dot_general(q, proto, (((1,), (1,)), ((), ())),
                         preferred_element_type=jnp.float32)
    d2 = qn + pn[None, :] - 2.0 * qp
    d_e = jnp.sqrt(jnp.clip(d2, 1e-12))

    z = -d_e
    zmax = jnp.max(z, axis=1, keepdims=True)
    iota_k = lax.broadcasted_iota(jnp.int32, (rows, K_WAY), 1)
    cand = jnp.where(z >= zmax, iota_k, K_WAY)
    pred = jnp.min(cand, axis=1, keepdims=True)
    onehot = jnp.where((iota_k == pred) & valid, 1.0, 0.0)

    cnt_scr[...] += jnp.sum(onehot, axis=0, keepdims=True)
    stats_scr[0] += lax.dot_general(onehot, q, (((0,), (0,)), ((), ())),
                                    preferred_element_type=jnp.float32)
    stats_scr[1] += lax.dot_general(onehot, qsq, (((0,), (0,)), ((), ())),
                                    preferred_element_type=jnp.float32)

    a = z - _logsumexp_rows(z)
    b = a - _logsumexp_rows(a)
    tgt_f = tgt.astype(jnp.float32)
    loss_ref[...] += -jnp.sum(tgt_f * b, keepdims=True).reshape(1, 1)


def _refresh_protos(sup_ref, stats_scr, cnt_scr, noise_ref, pnew_scr):
    sup = sup_ref[:NSUP, :].reshape(K_WAY, N_SHOT, D)
    proto = jnp.mean(sup, axis=1)
    cnt = cnt_scr[...].reshape(K_WAY, 1) + float(N_SHOT)
    s_sum = jnp.sum(sup, axis=1) + stats_scr[0]
    s_sq = jnp.sum(sup * sup, axis=1) + stats_scr[1]
    mean_c = s_sum / cnt
    var_c = (s_sq - cnt * mean_c * mean_c) / (cnt - 1.0)
    std_c = jnp.sqrt(jnp.clip(var_c, 1e-12))

    noise = noise_ref[...]
    delta = mean_c - proto
    u = delta * std_c
    v = std_c * std_c
    dn2 = jnp.sum(delta * delta, axis=1, keepdims=True)
    bdims = (((2,), (1,)), ((0,), (0,)))
    cross = lax.dot_general(noise, u, bdims,
                            preferred_element_type=jnp.float32)
    quad = lax.dot_general(noise * noise, v, bdims,
                           preferred_element_type=jnp.float32)
    d2_smp = dn2 + 2.0 * cross + quad

    dsup = sup - proto[:, None, :]
    d2_sup = jnp.sum(dsup * dsup, axis=2)
    d = jnp.sqrt(jnp.clip(jnp.concatenate([d2_sup, d2_smp], axis=1),
                          1e-12))

    di = d[:, :, None]
    dj = d[:, None, :]
    ii = lax.broadcasted_iota(jnp.int32, (K_WAY, NS, NS), 1)
    jj = lax.broadcasted_iota(jnp.int32, (K_WAY, NS, NS), 2)
    cmp = (dj < di) | ((dj == di) & (jj < ii))
    rank = jnp.sum(cmp.astype(jnp.float32), axis=2)
    sel = (rank < float(TOPK)).astype(jnp.float32)
    sel_sup = sel[:, :N_SHOT]
    sel_smp = sel[:, N_SHOT:]
    nsel = jnp.sum(sel_smp, axis=1, keepdims=True)

    sup_part = lax.dot_general(sel_sup, sup, (((1,), (1,)), ((0,), (0,))),
                               preferred_element_type=jnp.float32)
    noise_part = lax.dot_general(sel_smp, noise, (((1,), (1,)), ((0,), (0,))),
                                 preferred_element_type=jnp.float32)
    pnew = sup_part + nsel * mean_c + std_c * noise_part
    pnew_scr[:K_WAY] = pnew * (1.0 / TOPK)


def _softmax_rows(q, pnew, pn):
    qn = jnp.sum(q * q, axis=1, keepdims=True)
    qp = lax.dot_general(q, pnew, (((1,), (1,)), ((), ())),
                         preferred_element_type=jnp.float32)
    d2 = qn + pn[None, :] - 2.0 * qp
    z = -jnp.sqrt(jnp.clip(d2, 1e-12))
    m = jnp.max(z, axis=1, keepdims=True)
    e = jnp.exp(z - m)
    return e / jnp.sum(e, axis=1, keepdims=True)


def _body(q_ref, tailf_ref, tgt_ref, tgtt_ref, noise_ref,
          loss_ref, y_ref, yt_ref,
          proto_scr, stats_scr, cnt_scr, pnew_scr):
    i = pl.program_id(0)

    @pl.when(i == 0)
    def _init():
        stats_scr[...] = jnp.zeros_like(stats_scr)
        cnt_scr[...] = jnp.zeros_like(cnt_scr)
        loss_ref[...] = jnp.zeros_like(loss_ref)
        sup = q_ref[:NSUP, :].reshape(K_WAY, N_SHOT, D)
        proto_scr[:K_WAY] = jnp.mean(sup, axis=1)

    @pl.when(i < NT1)
    def _phase1():
        proto = proto_scr[:K_WAY]
        pn = jnp.sum(proto * proto, axis=1)
        CH = 4
        RW = QT // CH
        for c in range(CH):
            q = q_ref[pl.ds(c * RW, RW), :]
            r0 = (lax.broadcasted_iota(jnp.int32, (RW, 1), 0)
                  + i * QT + c * RW)
            valid = r0 >= NSUP
            _accumulate(q, tgt_ref[pl.ds(c * RW, RW), :], proto, pn, valid,
                        stats_scr, cnt_scr, loss_ref)

        @pl.when(i == NT1 - 1)
        def _tail():
            rloc = lax.broadcasted_iota(jnp.int32, (TAIL, 1), 0)
            tvalid = rloc < NSUP
            qt = jnp.where(tvalid, tailf_ref[...], 0.0)
            _accumulate(qt, tgtt_ref[...], proto, pn, tvalid,
                        stats_scr, cnt_scr, loss_ref)

    @pl.when(i == NT1)
    def _mid():
        _refresh_protos(q_ref, stats_scr, cnt_scr, noise_ref, pnew_scr)

    @pl.when(i >= NT1)
    def _phase2():
        pnew = pnew_scr[:K_WAY]
        pn = jnp.sum(pnew * pnew, axis=1)
        y_ref[...] = _softmax_rows(q_ref[...], pnew, pn)

        @pl.when(i == 2 * NT1 - 1)
        def _tail2():
            yt_ref[...] = _softmax_rows(tailf_ref[...], pnew, pn)


@jax.jit
def kernel(feat, label):
    tgt_oh = jax.nn.one_hot(label[1], K_WAY, dtype=jnp.int8)
    tgt_pad = jnp.pad(tgt_oh, ((NSUP, TAIL - NSUP), (0, 0)))

    loss_sum, y_main, y_tail = pl.pallas_call(
        _body,
        grid=(2 * NT1,),
        in_specs=[
            pl.BlockSpec((QT, D),
                         lambda i: (jnp.where(i < NT1, i, i - NT1), 0)),
            pl.BlockSpec((TAIL, D), lambda i: (NQ // TAIL, 0)),
            pl.BlockSpec((QT, K_WAY), lambda i: (jnp.minimum(i, NT1 - 1), 0)),
            pl.BlockSpec((TAIL, K_WAY), lambda i: (NQ // TAIL, 0)),
            pl.BlockSpec((K_WAY, N_AUG, D), lambda i: (0, 0, 0)),
        ],
        out_specs=[
            pl.BlockSpec((1, 1), lambda i: (0, 0)),
            pl.BlockSpec((QT, K_WAY),
                         lambda i: (jnp.where(i < NT1, 0, i - NT1), 0)),
            pl.BlockSpec((TAIL, K_WAY), lambda i: (0, 0)),
        ],
        out_shape=[
            jax.ShapeDtypeStruct((1, 1), jnp.float32),
            jax.ShapeDtypeStruct((NQ, K_WAY), jnp.float32),
            jax.ShapeDtypeStruct((TAIL, K_WAY), jnp.float32),
        ],
        scratch_shapes=[
            pltpu.VMEM((8, D), jnp.float32),
            pltpu.VMEM((2, K_WAY, D), jnp.float32),
            pltpu.VMEM((1, K_WAY), jnp.float32),
            pltpu.VMEM((8, D), jnp.float32),
        ],
    )(feat, feat, tgt_pad, tgt_pad, _NOISE)

    y_pred = jnp.concatenate(
        [lax.slice(y_main, (NSUP, 0), (NQ, K_WAY)),
         lax.slice(y_tail, (0, 0), (NSUP, K_WAY))], axis=0)
    loss = loss_sum[0, 0] / NQ
    return (y_pred, loss)


# speedup vs baseline: 1.2673x; 1.0015x over previous
import jax
import jax.numpy as jnp
from jax import lax
from jax.experimental import pallas as pl
from jax.experimental.pallas import tpu as pltpu

K_WAY = 5
N_SHOT = 5
NSUP = K_WAY * N_SHOT
D = 4096
NQ = 16384
N_AUG = 70
TOPK = 50
NS = K_WAY + N_AUG

QT = 1024
NT1 = NQ // QT
TAIL = 32

_NOISE = jax.random.normal(jax.random.key(42), (K_WAY, N_AUG, D),
                           dtype=jnp.float32)


def _logsumexp_rows(z):
    m = jnp.max(z, axis=1, keepdims=True)
    return m + jnp.log(jnp.sum(jnp.exp(z - m), axis=1, keepdims=True))


def _accumulate(q, tgt, proto, pn, valid, stats_scr, cnt_scr, loss_ref):
    rows = q.shape[0]
    qsq = q * q
    qn = jnp.sum(qsq, axis=1, keepdims=True)
    qp = lax.dot_general(q, proto, (((1,), (1,)), ((), ())),
                         preferred_element_type=jnp.float32)
    d2 = qn + pn[None, :] - 2.0 * qp
    d_e = jnp.sqrt(jnp.clip(d2, 1e-12))

    z = -d_e
    zmax = jnp.max(z, axis=1, keepdims=True)
    iota_k = lax.broadcasted_iota(jnp.int32, (rows, K_WAY), 1)
    cand = jnp.where(z >= zmax, iota_k, K_WAY)
    pred = jnp.min(cand, axis=1, keepdims=True)
    onehot = jnp.where((iota_k == pred) & valid, 1.0, 0.0)

    cnt_scr[...] += jnp.sum(onehot, axis=0, keepdims=True)
    stats_scr[0] += lax.dot_general(onehot, q, (((0,), (0,)), ((), ())),
                                    preferred_element_type=jnp.float32)
    stats_scr[1] += lax.dot_general(onehot, qsq, (((0,), (0,)), ((), ())),
                                    preferred_element_type=jnp.float32)

    a = z - _logsumexp_rows(z)
    b = a - _logsumexp_rows(a)
    tgt_f = tgt.astype(jnp.float32)
    loss_ref[...] += -jnp.sum(tgt_f * b, keepdims=True).reshape(1, 1)


def _refresh_protos(sup_ref, stats_scr, cnt_scr, noise_ref, pnew_scr):
    sup = sup_ref[:NSUP, :].reshape(K_WAY, N_SHOT, D)
    proto = jnp.mean(sup, axis=1)
    cnt = cnt_scr[...].reshape(K_WAY, 1) + float(N_SHOT)
    s_sum = jnp.sum(sup, axis=1) + stats_scr[0]
    s_sq = jnp.sum(sup * sup, axis=1) + stats_scr[1]
    mean_c = s_sum / cnt
    var_c = (s_sq - cnt * mean_c * mean_c) / (cnt - 1.0)
    std_c = jnp.sqrt(jnp.clip(var_c, 1e-12))

    noise = noise_ref[...]
    delta = mean_c - proto
    u = delta * std_c
    v = std_c * std_c
    dn2 = jnp.sum(delta * delta, axis=1, keepdims=True)
    bdims = (((2,), (1,)), ((0,), (0,)))
    cross = lax.dot_general(noise, u, bdims,
                            preferred_element_type=jnp.float32)
    quad = lax.dot_general(noise * noise, v, bdims,
                           preferred_element_type=jnp.float32)
    d2_smp = dn2 + 2.0 * cross + quad

    dsup = sup - proto[:, None, :]
    d2_sup = jnp.sum(dsup * dsup, axis=2)
    d = jnp.sqrt(jnp.clip(jnp.concatenate([d2_sup, d2_smp], axis=1),
                          1e-12))

    di = d[:, :, None]
    dj = d[:, None, :]
    ii = lax.broadcasted_iota(jnp.int32, (K_WAY, NS, NS), 1)
    jj = lax.broadcasted_iota(jnp.int32, (K_WAY, NS, NS), 2)
    cmp = (dj < di) | ((dj == di) & (jj < ii))
    rank = jnp.sum(cmp.astype(jnp.float32), axis=2)
    sel = (rank < float(TOPK)).astype(jnp.float32)
    sel_sup = sel[:, :N_SHOT]
    sel_smp = sel[:, N_SHOT:]
    nsel = jnp.sum(sel_smp, axis=1, keepdims=True)

    sup_part = lax.dot_general(sel_sup, sup, (((1,), (1,)), ((0,), (0,))),
                               preferred_element_type=jnp.float32)
    noise_part = lax.dot_general(sel_smp, noise, (((1,), (1,)), ((0,), (0,))),
                                 preferred_element_type=jnp.float32)
    pnew = sup_part + nsel * mean_c + std_c * noise_part
    pnew_scr[:K_WAY] = pnew * (1.0 / TOPK)


def _softmax_rows(q, pnew, pn):
    qn = jnp.sum(q * q, axis=1, keepdims=True)
    qp = lax.dot_general(q, pnew, (((1,), (1,)), ((), ())),
                         preferred_element_type=jnp.float32)
    d2 = qn + pn[None, :] - 2.0 * qp
    z = -jnp.sqrt(jnp.clip(d2, 1e-12))
    m = jnp.max(z, axis=1, keepdims=True)
    e = jnp.exp(z - m)
    return e / jnp.sum(e, axis=1, keepdims=True)


def _body(q_ref, tailf_ref, tgt_ref, tgtt_ref, noise_ref,
          loss_ref, y_ref, yt_ref,
          proto_scr, stats_scr, cnt_scr, pnew_scr):
    i = pl.program_id(0)

    @pl.when(i == 0)
    def _init():
        stats_scr[...] = jnp.zeros_like(stats_scr)
        cnt_scr[...] = jnp.zeros_like(cnt_scr)
        loss_ref[...] = jnp.zeros_like(loss_ref)
        sup = q_ref[:NSUP, :].reshape(K_WAY, N_SHOT, D)
        proto_scr[:K_WAY] = jnp.mean(sup, axis=1)

    @pl.when(i < NT1)
    def _phase1():
        proto = proto_scr[:K_WAY]
        pn = jnp.sum(proto * proto, axis=1)
        CH = 8
        RW = QT // CH
        for c in range(CH):
            q = q_ref[pl.ds(c * RW, RW), :]
            r0 = (lax.broadcasted_iota(jnp.int32, (RW, 1), 0)
                  + i * QT + c * RW)
            valid = r0 >= NSUP
            _accumulate(q, tgt_ref[pl.ds(c * RW, RW), :], proto, pn, valid,
                        stats_scr, cnt_scr, loss_ref)

        @pl.when(i == NT1 - 1)
        def _tail():
            rloc = lax.broadcasted_iota(jnp.int32, (TAIL, 1), 0)
            tvalid = rloc < NSUP
            qt = jnp.where(tvalid, tailf_ref[...], 0.0)
            _accumulate(qt, tgtt_ref[...], proto, pn, tvalid,
                        stats_scr, cnt_scr, loss_ref)

    @pl.when(i == NT1)
    def _mid():
        _refresh_protos(q_ref, stats_scr, cnt_scr, noise_ref, pnew_scr)

    @pl.when(i >= NT1)
    def _phase2():
        pnew = pnew_scr[:K_WAY]
        pn = jnp.sum(pnew * pnew, axis=1)
        y_ref[...] = _softmax_rows(q_ref[...], pnew, pn)

        @pl.when(i == 2 * NT1 - 1)
        def _tail2():
            yt_ref[...] = _softmax_rows(tailf_ref[...], pnew, pn)


@jax.jit
def kernel(feat, label):
    tgt_oh = jax.nn.one_hot(label[1], K_WAY, dtype=jnp.int8)
    tgt_pad = jnp.pad(tgt_oh, ((NSUP, TAIL - NSUP), (0, 0)))

    loss_sum, y_main, y_tail = pl.pallas_call(
        _body,
        grid=(2 * NT1,),
        in_specs=[
            pl.BlockSpec((QT, D),
                         lambda i: (jnp.where(i < NT1, i, i - NT1), 0)),
            pl.BlockSpec((TAIL, D), lambda i: (NQ // TAIL, 0)),
            pl.BlockSpec((QT, K_WAY), lambda i: (jnp.minimum(i, NT1 - 1), 0)),
            pl.BlockSpec((TAIL, K_WAY), lambda i: (NQ // TAIL, 0)),
            pl.BlockSpec((K_WAY, N_AUG, D), lambda i: (0, 0, 0)),
        ],
        out_specs=[
            pl.BlockSpec((1, 1), lambda i: (0, 0)),
            pl.BlockSpec((QT, K_WAY),
                         lambda i: (jnp.where(i < NT1, 0, i - NT1), 0)),
            pl.BlockSpec((TAIL, K_WAY), lambda i: (0, 0)),
        ],
        out_shape=[
            jax.ShapeDtypeStruct((1, 1), jnp.float32),
            jax.ShapeDtypeStruct((NQ, K_WAY), jnp.float32),
            jax.ShapeDtypeStruct((TAIL, K_WAY), jnp.float32),
        ],
        scratch_shapes=[
            pltpu.VMEM((8, D), jnp.float32),
            pltpu.VMEM((2, K_WAY, D), jnp.float32),
            pltpu.VMEM((1, K_WAY), jnp.float32),
            pltpu.VMEM((8, D), jnp.float32),
        ],
    )(feat, feat, tgt_pad, tgt_pad, _NOISE)

    y_pred = jnp.concatenate(
        [lax.slice(y_main, (NSUP, 0), (NQ, K_WAY)),
         lax.slice(y_tail, (0, 0), (NSUP, K_WAY))], axis=0)
    loss = loss_sum[0, 0] / NQ
    return (y_pred, loss)
